# Initial kernel scaffold; baseline (speedup 1.0000x reference)
import jax
import jax.numpy as jnp
from jax.experimental import pallas as pl


def kernel(x, edge_index, W1, att_src1, att_dst1, b1, gamma1, beta1, W2, att_src2, att_dst2, b2, Wd, bd):
    raise NotImplementedError("write your pallas kernel here")


# speedup vs baseline: 9.3513x; 9.3513x over previous
import dataclasses
import functools

import jax
import jax.numpy as jnp
from jax import lax
from jax.experimental import pallas as pl
from jax.experimental.pallas import tpu as pltpu
from jax.experimental.pallas import tpu_sc as plsc

_N = 10000
_NP = 10112
_NW = 32
_NCHUNK = 81
_CHW = 128
_EP = _NW * _NCHUNK * _CHW


def _compiler_params():
    cp = pltpu.CompilerParams()
    if "needs_layout_passes" in pltpu.CompilerParams.__dataclass_fields__:
        cp = dataclasses.replace(cp, needs_layout_passes=False)
    return cp


def _sc_attention(srcr, dstr, asrc_p, adst_p):
    nr = _NP // 16
    mesh = plsc.VectorSubcoreMesh(core_axis_name="c", subcore_axis_name="s")
    zeros_den = jnp.zeros((_NP, 16), jnp.float32)

    @functools.partial(
        pl.kernel,
        compiler_params=_compiler_params(),
        out_type=(
            jax.ShapeDtypeStruct((_NW, _NCHUNK, _CHW), jnp.float32),
            jax.ShapeDtypeStruct((2, _NP, 16), jnp.float32),
        ),
        mesh=mesh,
        scratch_types=[
            pltpu.VMEM((_NP,), jnp.float32),
            pltpu.VMEM((_NP,), jnp.float32),
            pltpu.VMEM((_CHW,), jnp.int32),
            pltpu.VMEM((_CHW,), jnp.int32),
            pltpu.VMEM((_CHW,), jnp.float32),
            pltpu.VMEM((_CHW, 16), jnp.float32),
            pltpu.VMEM_SHARED((_NP, 16), jnp.float32),
        ],
    )
    def k(srch, dsth, asrch, adsth, zdh, ee_out, den_out,
          asv, adv, srcj, dstj, eec, denrows, dens):
        c = lax.axis_index("c")
        s = lax.axis_index("s")
        w = c * 16 + s
        base = s * nr

        pltpu.sync_copy(asrch, asv)
        pltpu.sync_copy(adsth, adv)
        pltpu.sync_copy(zdh.at[pl.ds(base, nr)], dens.at[pl.ds(base, nr)])
        plsc.subcore_barrier()

        @pl.loop(0, _NCHUNK)
        def _(j):
            pltpu.sync_copy(srch.at[w, j], srcj)
            pltpu.sync_copy(dsth.at[w, j], dstj)
            for g in range(8):
                sv = srcj[pl.ds(g * 16, 16)]
                dv = dstj[pl.ds(g * 16, 16)]
                e = plsc.load_gather(asv, [sv]) + plsc.load_gather(adv, [dv])
                e = jnp.where(e >= 0.0, e, e * jnp.float32(0.2))
                eec[pl.ds(g * 16, 16)] = jnp.exp(e)

            @pl.loop(0, _CHW)
            def _(r):
                bc = plsc.load_gather(eec, [jnp.full((16,), r, jnp.int32)])
                denrows[r, :] = bc

            pltpu.sync_copy(denrows, dens.at[dstj], add=True)
            pltpu.sync_copy(eec, ee_out.at[w, j])

        plsc.subcore_barrier()
        pltpu.sync_copy(dens.at[pl.ds(base, nr)], den_out.at[c, pl.ds(base, nr)])

    return k(srcr, dstr, asrc_p, adst_p, zeros_den)


def _sc_aggregate(tables, srcr, dstr, eer):
    DC = tables[0].shape[1]
    n_t = len(tables)
    nr = _NP // 16
    mesh = plsc.VectorSubcoreMesh(core_axis_name="c", subcore_axis_name="s")
    zeros_acc = jnp.zeros((_NP, DC), jnp.float32)

    @functools.partial(
        pl.kernel,
        compiler_params=_compiler_params(),
        out_type=jax.ShapeDtypeStruct((2, n_t, _NP, DC), jnp.float32),
        mesh=mesh,
        scratch_types=[
            pltpu.VMEM((_CHW,), jnp.int32),
            pltpu.VMEM((_CHW,), jnp.int32),
            pltpu.VMEM((_CHW,), jnp.float32),
            pltpu.VMEM((_CHW, DC), jnp.float32),
            pltpu.VMEM_SHARED((_NP, DC), jnp.float32),
            pltpu.SemaphoreType.DMA,
        ],
    )
    def k(*refs):
        t_refs = refs[:n_t]
        (srch, dsth, eeh, zah, acc_out,
         srcj, dstj, eej, rows, accs, sem) = refs[n_t:]
        c = lax.axis_index("c")
        s = lax.axis_index("s")
        w = c * 16 + s
        base = s * nr

        pltpu.sync_copy(zah.at[pl.ds(base, nr)], accs.at[pl.ds(base, nr)])
        plsc.subcore_barrier()

        for cc in range(n_t):
            tbl = t_refs[cc]

            @pl.loop(0, _NCHUNK)
            def _(j, tbl=tbl):
                pltpu.sync_copy(srch.at[w, j], srcj)
                pltpu.sync_copy(dsth.at[w, j], dstj)
                pltpu.sync_copy(eeh.at[w, j], eej)
                pltpu.async_copy(tbl.at[srcj], rows, sem).wait()

                @pl.loop(0, _CHW)
                def _(r):
                    bc = plsc.load_gather(eej, [jnp.full((16,), r, jnp.int32)])
                    for d in range(DC // 16):
                        sl = pl.ds(d * 16, 16)
                        rows[r, sl] = rows[r, sl] * bc

                pltpu.sync_copy(rows, accs.at[dstj], add=True)

            plsc.subcore_barrier()
            pltpu.sync_copy(accs.at[pl.ds(base, nr)],
                            acc_out.at[c, cc, pl.ds(base, nr)])
            if cc + 1 < n_t:
                pltpu.sync_copy(zah.at[pl.ds(base, nr)],
                                accs.at[pl.ds(base, nr)])
                plsc.subcore_barrier()

    return k(*tables, srcr, dstr, eer, zeros_acc)


def _k1(x, W1, att_s, att_d):
    R = 1000

    def body(x_ref, w_ref, as_ref, ad_ref, h_ref, s_ref, d_ref):
        h = jnp.dot(x_ref[...], w_ref[...], preferred_element_type=jnp.float32)
        h_ref[...] = h
        s_ref[...] = (h * as_ref[...]).sum(axis=1, keepdims=True)
        d_ref[...] = (h * ad_ref[...]).sum(axis=1, keepdims=True)

    return pl.pallas_call(
        body,
        grid=(10,),
        in_specs=[
            pl.BlockSpec((R, 128), lambda i: (i, 0)),
            pl.BlockSpec((128, 512), lambda i: (0, 0)),
            pl.BlockSpec((1, 512), lambda i: (0, 0)),
            pl.BlockSpec((1, 512), lambda i: (0, 0)),
        ],
        out_specs=[
            pl.BlockSpec((R, 512), lambda i: (i, 0)),
            pl.BlockSpec((R, 1), lambda i: (i, 0)),
            pl.BlockSpec((R, 1), lambda i: (i, 0)),
        ],
        out_shape=[
            jax.ShapeDtypeStruct((_N, 512), jnp.float32),
            jax.ShapeDtypeStruct((_N, 1), jnp.float32),
            jax.ShapeDtypeStruct((_N, 1), jnp.float32),
        ],
    )(x, W1, att_s, att_d)


def _k2a(acc, den, b1):
    R = 1000

    def body(a_ref, d_ref, b_ref, o_ref, st_ref):
        i = pl.program_id(0)
        a = a_ref[...]
        asum = a[0] + a[1]
        cat = jnp.concatenate([asum[c] for c in range(4)], axis=1)
        dn = d_ref[0, :, 0] + d_ref[1, :, 0]
        o = cat / (dn[:, None] + 1e-16) + b_ref[...]
        o_ref[...] = o
        st = jnp.concatenate(
            [o.sum(axis=0, keepdims=True), (o * o).sum(axis=0, keepdims=True)],
            axis=0)

        @pl.when(i == 0)
        def _():
            st_ref[...] = st

        @pl.when(i != 0)
        def _():
            st_ref[...] += st

    return pl.pallas_call(
        body,
        grid=(10,),
        in_specs=[
            pl.BlockSpec((2, 4, R, 128), lambda i: (0, 0, i, 0)),
            pl.BlockSpec((2, R, 16), lambda i: (0, i, 0)),
            pl.BlockSpec((1, 512), lambda i: (0, 0)),
        ],
        out_specs=[
            pl.BlockSpec((R, 512), lambda i: (i, 0)),
            pl.BlockSpec((2, 512), lambda i: (0, 0)),
        ],
        out_shape=[
            jax.ShapeDtypeStruct((_N, 512), jnp.float32),
            jax.ShapeDtypeStruct((2, 512), jnp.float32),
        ],
    )(acc, den, b1)


def _k2b(o1, st, gamma, beta, W2p, as2, ad2):
    R = 1000

    def body(o_ref, st_ref, g_ref, b_ref, w_ref, s_ref, d_ref,
             h2_ref, s2_ref, d2_ref):
        st = st_ref[...]
        mu = st[0:1] / _N
        var = st[1:2] / _N - mu * mu
        xb = (o_ref[...] - mu) * lax.rsqrt(var + 1e-5) * g_ref[...] + b_ref[...]
        h = jnp.where(xb > 0, xb, jnp.exp(jnp.minimum(xb, 0.0)) - 1.0)
        h2 = jnp.dot(h, w_ref[...], preferred_element_type=jnp.float32)
        h2_ref[...] = h2
        s2_ref[...] = (h2 * s_ref[...]).sum(axis=1, keepdims=True)
        d2_ref[...] = (h2 * d_ref[...]).sum(axis=1, keepdims=True)

    return pl.pallas_call(
        body,
        grid=(10,),
        in_specs=[
            pl.BlockSpec((R, 512), lambda i: (i, 0)),
            pl.BlockSpec((2, 512), lambda i: (0, 0)),
            pl.BlockSpec((1, 512), lambda i: (0, 0)),
            pl.BlockSpec((1, 512), lambda i: (0, 0)),
            pl.BlockSpec((512, 128), lambda i: (0, 0)),
            pl.BlockSpec((1, 128), lambda i: (0, 0)),
            pl.BlockSpec((1, 128), lambda i: (0, 0)),
        ],
        out_specs=[
            pl.BlockSpec((R, 128), lambda i: (i, 0)),
            pl.BlockSpec((R, 1), lambda i: (i, 0)),
            pl.BlockSpec((R, 1), lambda i: (i, 0)),
        ],
        out_shape=[
            jax.ShapeDtypeStruct((_N, 128), jnp.float32),
            jax.ShapeDtypeStruct((_N, 1), jnp.float32),
            jax.ShapeDtypeStruct((_N, 1), jnp.float32),
        ],
    )(o1, st, gamma, beta, W2p, as2, ad2)


def _k3(acc2, den2, b2p, Wdp, bd):
    R = 1000

    def body(a_ref, d_ref, b_ref, w_ref, bd_ref, z_ref, x_ref):
        a = a_ref[0] + a_ref[1]
        dn = d_ref[0, :, 0] + d_ref[1, :, 0]
        z = a / (dn[:, None] + 1e-16) + b_ref[...]
        z_ref[...] = z
        x_ref[...] = (jnp.dot(z, w_ref[...], preferred_element_type=jnp.float32)
                      + bd_ref[...])

    return pl.pallas_call(
        body,
        grid=(10,),
        in_specs=[
            pl.BlockSpec((2, R, 128), lambda i: (0, i, 0)),
            pl.BlockSpec((2, R, 16), lambda i: (0, i, 0)),
            pl.BlockSpec((1, 128), lambda i: (0, 0)),
            pl.BlockSpec((128, 128), lambda i: (0, 0)),
            pl.BlockSpec((1, 128), lambda i: (0, 0)),
        ],
        out_specs=[
            pl.BlockSpec((R, 128), lambda i: (i, 0)),
            pl.BlockSpec((R, 128), lambda i: (i, 0)),
        ],
        out_shape=[
            jax.ShapeDtypeStruct((_N, 128), jnp.float32),
            jax.ShapeDtypeStruct((_N, 128), jnp.float32),
        ],
    )(acc2, den2, b2p, Wdp, bd)


def kernel(x, edge_index, W1, att_src1, att_dst1, b1, gamma1, beta1,
           W2, att_src2, att_dst2, b2, Wd, bd):
    loop = jnp.arange(_N, dtype=jnp.int32)
    src = jnp.concatenate([edge_index[0].astype(jnp.int32), loop])
    dst = jnp.concatenate([edge_index[1].astype(jnp.int32), loop])
    pad = _EP - src.shape[0]
    src = jnp.concatenate([src, jnp.zeros((pad,), jnp.int32)])
    dst = jnp.concatenate([dst, jnp.full((pad,), _N, jnp.int32)])
    srcr = src.reshape(_NW, _NCHUNK, _CHW)
    dstr = dst.reshape(_NW, _NCHUNK, _CHW)
    zpad = jnp.zeros((_NP - _N,), jnp.float32)

    h1, a_s1, a_d1 = _k1(x, W1, att_src1, att_dst1)
    asp = jnp.concatenate([a_s1[:, 0], zpad])
    adp = jnp.concatenate([a_d1[:, 0], zpad])
    h1c = h1.reshape(_N, 4, 128).transpose(1, 0, 2)
    tables1 = tuple(h1c[c] for c in range(4))
    ee1, den1 = _sc_attention(srcr, dstr, asp, adp)
    acc1 = _sc_aggregate(tables1, srcr, dstr, ee1)

    o1, st1 = _k2a(acc1, den1, b1.reshape(1, 512))
    W2p = jnp.concatenate([W2, jnp.zeros((512, 98), jnp.float32)], axis=1)
    as2p = jnp.concatenate([att_src2, jnp.zeros((1, 98), jnp.float32)], axis=1)
    ad2p = jnp.concatenate([att_dst2, jnp.zeros((1, 98), jnp.float32)], axis=1)
    h2p, a_s2, a_d2 = _k2b(o1, st1, gamma1.reshape(1, 512),
                           beta1.reshape(1, 512), W2p, as2p, ad2p)
    asp2 = jnp.concatenate([a_s2[:, 0], zpad])
    adp2 = jnp.concatenate([a_d2[:, 0], zpad])
    ee2, den2 = _sc_attention(srcr, dstr, asp2, adp2)
    acc2 = _sc_aggregate((h2p,), srcr, dstr, ee2)

    b2p = jnp.concatenate([b2, jnp.zeros((98,), jnp.float32)]).reshape(1, 128)
    Wdp = jnp.concatenate([Wd, jnp.zeros((98, 128), jnp.float32)], axis=0)
    zp, xr = _k3(acc2.reshape(2, _NP, 128), den2, b2p, Wdp, bd.reshape(1, 128))
    return zp[:, :30], xr


# speedup vs baseline: 10.6160x; 1.1352x over previous
import dataclasses
import functools

import jax
import jax.numpy as jnp
from jax import lax
from jax.experimental import pallas as pl
from jax.experimental.pallas import tpu as pltpu
from jax.experimental.pallas import tpu_sc as plsc

_N = 10000
_NP = 10112
_NW = 32
_NCHUNK = 81
_CHW = 128
_EP = _NW * _NCHUNK * _CHW


def _compiler_params():
    cp = pltpu.CompilerParams()
    if "needs_layout_passes" in pltpu.CompilerParams.__dataclass_fields__:
        cp = dataclasses.replace(cp, needs_layout_passes=False)
    return cp


def _sc_attention(srcr, dstr, asrc_p, adst_p):
    nr = _NP // 16
    mesh = plsc.VectorSubcoreMesh(core_axis_name="c", subcore_axis_name="s")
    zeros_den = jnp.zeros((_NP, 16), jnp.float32)

    @functools.partial(
        pl.kernel,
        compiler_params=_compiler_params(),
        out_type=(
            jax.ShapeDtypeStruct((_NW, _NCHUNK, _CHW), jnp.float32),
            jax.ShapeDtypeStruct((2, _NP, 16), jnp.float32),
        ),
        mesh=mesh,
        scratch_types=[
            pltpu.VMEM((_NP,), jnp.float32),
            pltpu.VMEM((_NP,), jnp.float32),
            pltpu.VMEM((_CHW,), jnp.int32),
            pltpu.VMEM((_CHW,), jnp.int32),
            pltpu.VMEM((_CHW,), jnp.float32),
            pltpu.VMEM((_CHW, 16), jnp.float32),
            pltpu.VMEM_SHARED((_NP, 16), jnp.float32),
        ],
    )
    def k(srch, dsth, asrch, adsth, zdh, ee_out, den_out,
          asv, adv, srcj, dstj, eec, denrows, dens):
        c = lax.axis_index("c")
        s = lax.axis_index("s")
        w = c * 16 + s
        base = s * nr

        pltpu.sync_copy(asrch, asv)
        pltpu.sync_copy(adsth, adv)
        pltpu.sync_copy(zdh.at[pl.ds(base, nr)], dens.at[pl.ds(base, nr)])
        plsc.subcore_barrier()

        @pl.loop(0, _NCHUNK)
        def _(j):
            pltpu.sync_copy(srch.at[w, j], srcj)
            pltpu.sync_copy(dsth.at[w, j], dstj)
            for g in range(8):
                sv = srcj[pl.ds(g * 16, 16)]
                dv = dstj[pl.ds(g * 16, 16)]
                e = plsc.load_gather(asv, [sv]) + plsc.load_gather(adv, [dv])
                e = jnp.where(e >= 0.0, e, e * jnp.float32(0.2))
                eec[pl.ds(g * 16, 16)] = jnp.exp(e)

            @plsc.parallel_loop(0, _CHW, unroll=8)
            def _(r):
                bc = plsc.load_gather(eec, [jnp.full((16,), r, jnp.int32)])
                denrows[r, :] = bc

            pltpu.sync_copy(denrows, dens.at[dstj], add=True)
            pltpu.sync_copy(eec, ee_out.at[w, j])

        plsc.subcore_barrier()
        pltpu.sync_copy(dens.at[pl.ds(base, nr)], den_out.at[c, pl.ds(base, nr)])

    return k(srcr, dstr, asrc_p, adst_p, zeros_den)


def _sc_aggregate(tables, srcr, dstr, eer):
    DC = tables[0].shape[1]
    n_t = len(tables)
    nr = _NP // 16
    mesh = plsc.VectorSubcoreMesh(core_axis_name="c", subcore_axis_name="s")
    zeros_acc = jnp.zeros((_NP, DC), jnp.float32)

    @functools.partial(
        pl.kernel,
        compiler_params=_compiler_params(),
        out_type=jax.ShapeDtypeStruct((2, n_t, _NP, DC), jnp.float32),
        mesh=mesh,
        scratch_types=[
            pltpu.VMEM((_CHW,), jnp.int32),
            pltpu.VMEM((_CHW,), jnp.int32),
            pltpu.VMEM((_CHW,), jnp.float32),
            pltpu.VMEM((_CHW, DC), jnp.float32),
            pltpu.VMEM_SHARED((_NP, DC), jnp.float32),
            pltpu.SemaphoreType.DMA,
        ],
    )
    def k(*refs):
        t_refs = refs[:n_t]
        (srch, dsth, eeh, zah, acc_out,
         srcj, dstj, eej, rows, accs, sem) = refs[n_t:]
        c = lax.axis_index("c")
        s = lax.axis_index("s")
        w = c * 16 + s
        base = s * nr

        pltpu.sync_copy(zah.at[pl.ds(base, nr)], accs.at[pl.ds(base, nr)])
        plsc.subcore_barrier()

        for cc in range(n_t):
            tbl = t_refs[cc]

            @pl.loop(0, _NCHUNK)
            def _(j, tbl=tbl):
                pltpu.sync_copy(srch.at[w, j], srcj)
                pltpu.sync_copy(dsth.at[w, j], dstj)
                pltpu.sync_copy(eeh.at[w, j], eej)
                pltpu.async_copy(tbl.at[srcj], rows, sem).wait()

                @plsc.parallel_loop(0, _CHW, unroll=4)
                def _(r):
                    bc = plsc.load_gather(eej, [jnp.full((16,), r, jnp.int32)])
                    for d in range(DC // 16):
                        sl = pl.ds(d * 16, 16)
                        rows[r, sl] = rows[r, sl] * bc

                pltpu.sync_copy(rows, accs.at[dstj], add=True)

            plsc.subcore_barrier()
            pltpu.sync_copy(accs.at[pl.ds(base, nr)],
                            acc_out.at[c, cc, pl.ds(base, nr)])
            if cc + 1 < n_t:
                pltpu.sync_copy(zah.at[pl.ds(base, nr)],
                                accs.at[pl.ds(base, nr)])
                plsc.subcore_barrier()

    return k(*tables, srcr, dstr, eer, zeros_acc)


def _k1(x, W1, att_s, att_d):
    R = 1000

    def body(x_ref, w_ref, as_ref, ad_ref, h_ref, s_ref, d_ref):
        h = jnp.dot(x_ref[...], w_ref[...], preferred_element_type=jnp.float32)
        h_ref[...] = h
        s_ref[...] = (h * as_ref[...]).sum(axis=1, keepdims=True)
        d_ref[...] = (h * ad_ref[...]).sum(axis=1, keepdims=True)

    return pl.pallas_call(
        body,
        grid=(10,),
        in_specs=[
            pl.BlockSpec((R, 128), lambda i: (i, 0)),
            pl.BlockSpec((128, 512), lambda i: (0, 0)),
            pl.BlockSpec((1, 512), lambda i: (0, 0)),
            pl.BlockSpec((1, 512), lambda i: (0, 0)),
        ],
        out_specs=[
            pl.BlockSpec((R, 512), lambda i: (i, 0)),
            pl.BlockSpec((R, 1), lambda i: (i, 0)),
            pl.BlockSpec((R, 1), lambda i: (i, 0)),
        ],
        out_shape=[
            jax.ShapeDtypeStruct((_N, 512), jnp.float32),
            jax.ShapeDtypeStruct((_N, 1), jnp.float32),
            jax.ShapeDtypeStruct((_N, 1), jnp.float32),
        ],
    )(x, W1, att_s, att_d)


def _k2a(acc, den, b1):
    R = 1000

    def body(a_ref, d_ref, b_ref, o_ref, st_ref):
        i = pl.program_id(0)
        a = a_ref[...]
        asum = a[0] + a[1]
        cat = jnp.concatenate([asum[c] for c in range(4)], axis=1)
        dn = d_ref[0, :, 0] + d_ref[1, :, 0]
        o = cat / (dn[:, None] + 1e-16) + b_ref[...]
        o_ref[...] = o
        st = jnp.concatenate(
            [o.sum(axis=0, keepdims=True), (o * o).sum(axis=0, keepdims=True)],
            axis=0)

        @pl.when(i == 0)
        def _():
            st_ref[...] = st

        @pl.when(i != 0)
        def _():
            st_ref[...] += st

    return pl.pallas_call(
        body,
        grid=(10,),
        in_specs=[
            pl.BlockSpec((2, 4, R, 128), lambda i: (0, 0, i, 0)),
            pl.BlockSpec((2, R, 16), lambda i: (0, i, 0)),
            pl.BlockSpec((1, 512), lambda i: (0, 0)),
        ],
        out_specs=[
            pl.BlockSpec((R, 512), lambda i: (i, 0)),
            pl.BlockSpec((2, 512), lambda i: (0, 0)),
        ],
        out_shape=[
            jax.ShapeDtypeStruct((_N, 512), jnp.float32),
            jax.ShapeDtypeStruct((2, 512), jnp.float32),
        ],
    )(acc, den, b1)


def _k2b(o1, st, gamma, beta, W2p, as2, ad2):
    R = 1000

    def body(o_ref, st_ref, g_ref, b_ref, w_ref, s_ref, d_ref,
             h2_ref, s2_ref, d2_ref):
        st = st_ref[...]
        mu = st[0:1] / _N
        var = st[1:2] / _N - mu * mu
        xb = (o_ref[...] - mu) * lax.rsqrt(var + 1e-5) * g_ref[...] + b_ref[...]
        h = jnp.where(xb > 0, xb, jnp.exp(jnp.minimum(xb, 0.0)) - 1.0)
        h2 = jnp.dot(h, w_ref[...], preferred_element_type=jnp.float32)
        h2_ref[...] = h2
        s2_ref[...] = (h2 * s_ref[...]).sum(axis=1, keepdims=True)
        d2_ref[...] = (h2 * d_ref[...]).sum(axis=1, keepdims=True)

    return pl.pallas_call(
        body,
        grid=(10,),
        in_specs=[
            pl.BlockSpec((R, 512), lambda i: (i, 0)),
            pl.BlockSpec((2, 512), lambda i: (0, 0)),
            pl.BlockSpec((1, 512), lambda i: (0, 0)),
            pl.BlockSpec((1, 512), lambda i: (0, 0)),
            pl.BlockSpec((512, 128), lambda i: (0, 0)),
            pl.BlockSpec((1, 128), lambda i: (0, 0)),
            pl.BlockSpec((1, 128), lambda i: (0, 0)),
        ],
        out_specs=[
            pl.BlockSpec((R, 128), lambda i: (i, 0)),
            pl.BlockSpec((R, 1), lambda i: (i, 0)),
            pl.BlockSpec((R, 1), lambda i: (i, 0)),
        ],
        out_shape=[
            jax.ShapeDtypeStruct((_N, 128), jnp.float32),
            jax.ShapeDtypeStruct((_N, 1), jnp.float32),
            jax.ShapeDtypeStruct((_N, 1), jnp.float32),
        ],
    )(o1, st, gamma, beta, W2p, as2, ad2)


def _k3(acc2, den2, b2p, Wdp, bd):
    R = 1000

    def body(a_ref, d_ref, b_ref, w_ref, bd_ref, z_ref, x_ref):
        a = a_ref[0] + a_ref[1]
        dn = d_ref[0, :, 0] + d_ref[1, :, 0]
        z = a / (dn[:, None] + 1e-16) + b_ref[...]
        z_ref[...] = z
        x_ref[...] = (jnp.dot(z, w_ref[...], preferred_element_type=jnp.float32)
                      + bd_ref[...])

    return pl.pallas_call(
        body,
        grid=(10,),
        in_specs=[
            pl.BlockSpec((2, R, 128), lambda i: (0, i, 0)),
            pl.BlockSpec((2, R, 16), lambda i: (0, i, 0)),
            pl.BlockSpec((1, 128), lambda i: (0, 0)),
            pl.BlockSpec((128, 128), lambda i: (0, 0)),
            pl.BlockSpec((1, 128), lambda i: (0, 0)),
        ],
        out_specs=[
            pl.BlockSpec((R, 128), lambda i: (i, 0)),
            pl.BlockSpec((R, 128), lambda i: (i, 0)),
        ],
        out_shape=[
            jax.ShapeDtypeStruct((_N, 128), jnp.float32),
            jax.ShapeDtypeStruct((_N, 128), jnp.float32),
        ],
    )(acc2, den2, b2p, Wdp, bd)


def kernel(x, edge_index, W1, att_src1, att_dst1, b1, gamma1, beta1,
           W2, att_src2, att_dst2, b2, Wd, bd):
    loop = jnp.arange(_N, dtype=jnp.int32)
    src = jnp.concatenate([edge_index[0].astype(jnp.int32), loop])
    dst = jnp.concatenate([edge_index[1].astype(jnp.int32), loop])
    pad = _EP - src.shape[0]
    src = jnp.concatenate([src, jnp.zeros((pad,), jnp.int32)])
    dst = jnp.concatenate([dst, jnp.full((pad,), _N, jnp.int32)])
    srcr = src.reshape(_NW, _NCHUNK, _CHW)
    dstr = dst.reshape(_NW, _NCHUNK, _CHW)
    zpad = jnp.zeros((_NP - _N,), jnp.float32)

    h1, a_s1, a_d1 = _k1(x, W1, att_src1, att_dst1)
    asp = jnp.concatenate([a_s1[:, 0], zpad])
    adp = jnp.concatenate([a_d1[:, 0], zpad])
    h1c = h1.reshape(_N, 4, 128).transpose(1, 0, 2)
    tables1 = tuple(h1c[c] for c in range(4))
    ee1, den1 = _sc_attention(srcr, dstr, asp, adp)
    acc1 = _sc_aggregate(tables1, srcr, dstr, ee1)

    o1, st1 = _k2a(acc1, den1, b1.reshape(1, 512))
    W2p = jnp.concatenate([W2, jnp.zeros((512, 98), jnp.float32)], axis=1)
    as2p = jnp.concatenate([att_src2, jnp.zeros((1, 98), jnp.float32)], axis=1)
    ad2p = jnp.concatenate([att_dst2, jnp.zeros((1, 98), jnp.float32)], axis=1)
    h2p, a_s2, a_d2 = _k2b(o1, st1, gamma1.reshape(1, 512),
                           beta1.reshape(1, 512), W2p, as2p, ad2p)
    asp2 = jnp.concatenate([a_s2[:, 0], zpad])
    adp2 = jnp.concatenate([a_d2[:, 0], zpad])
    ee2, den2 = _sc_attention(srcr, dstr, asp2, adp2)
    acc2 = _sc_aggregate((h2p,), srcr, dstr, ee2)

    b2p = jnp.concatenate([b2, jnp.zeros((98,), jnp.float32)]).reshape(1, 128)
    Wdp = jnp.concatenate([Wd, jnp.zeros((98, 128), jnp.float32)], axis=0)
    zp, xr = _k3(acc2.reshape(2, _NP, 128), den2, b2p, Wdp, bd.reshape(1, 128))
    return zp[:, :30], xr
